# baseline (speedup 1.0000x reference)
import functools

import jax
import jax.numpy as jnp
from jax import lax
from jax.experimental import pallas as pl
from jax.experimental.pallas import tpu as pltpu, tpu_sc as plsc

N = 1000000
NZ = 64
B = 16384

_info = plsc.get_sparse_core_info()
_NC, _NS, _L = _info.num_cores, _info.num_subcores, _info.num_lanes
_NW = _NC * _NS
_NBLK = (N + 127) // 128
_BPWF = _NBLK // _NW
_REM = _NBLK - _BPWF * _NW
_CB = 1
_ICH = 1024
_SROWS = 32
_WLSZ = 2048


def _make_gather():
    mesh = plsc.VectorSubcoreMesh(core_axis_name="c", subcore_axis_name="s")

    @functools.partial(
        pl.kernel,
        mesh=mesh,
        out_type=jax.ShapeDtypeStruct((B, NZ), jnp.float32),
        scratch_types=[
            pltpu.VMEM((_ICH,), jnp.int32),
            pltpu.VMEM((_WLSZ + _L,), jnp.int32),
            pltpu.VMEM((_WLSZ + _L,), jnp.int32),
            pltpu.VMEM((12, NZ, _CB * 128), jnp.float32),
            pltpu.VMEM((_SROWS, NZ), jnp.float32),
            pltpu.SMEM((_BPWF + 2,), jnp.int32),
            pltpu.SMEM((_BPWF + 2,), jnp.int32),
            pltpu.SMEM((_BPWF + 2,), jnp.int32),
            [pltpu.SemaphoreType.DMA] * 12,
            pltpu.SemaphoreType.DMA,
        ],
        compiler_params=pltpu.CompilerParams(needs_layout_passes=False),
    )
    def gather_kernel(idx_hbm, table_hbm, out_hbm, idxb_v, wl_v, wl2_v,
                      ring_v, stage_v, cnt_s, start_s, cur_s,
                      sems, sem_o):
        wid = lax.axis_index("s") * _NC + lax.axis_index("c")
        lo = _BPWF * wid + jnp.minimum(wid, _REM)
        nb = _BPWF + jnp.where(wid < _REM, 1, 0)
        nch = (nb + _CB - 1) // _CB
        jota = lax.iota(jnp.int32, _L)

        def fetch_t(t):
            gs = lo + jnp.minimum(t * _CB, nb - _CB)
            b0 = pl.multiple_of(gs * 128, 128)
            buf = lax.rem(t, 12)
            for bi, sem in enumerate(sems):
                @pl.when(buf == bi)
                def _(bi=bi, sem=sem):
                    pltpu.async_copy(
                        table_hbm.at[:, pl.ds(b0, _CB * 128)],
                        ring_v.at[bi], sem)

        def wait_t(t):
            buf = lax.rem(t, 12)
            for bi, sem in enumerate(sems):
                @pl.when(buf == bi)
                def _(bi=bi, sem=sem):
                    pltpu.make_async_copy(
                        table_hbm.at[:, pl.ds(0, _CB * 128)],
                        ring_v.at[bi], sem).wait()

        for _t in range(12):
            fetch_t(_t)

        def fa_chunk(c, off):
            pltpu.sync_copy(idx_hbm.at[pl.ds(c * _ICH, _ICH)], idxb_v)

            def fa_vec(g, off):
                v = idxb_v[pl.ds(g * _L, _L)]
                blk = v >> 7
                m = (blk >= lo) & (blk < lo + nb)
                i_vec = c * _ICH + g * _L + jota
                e = (i_vec << 15) | ((v & 127) << 8) | (blk - lo)
                cnt = plsc.all_reduce_population_count(m)
                plsc.store_compressed(wl_v.at[pl.ds(off, _L)], e, mask=m)
                return jnp.minimum(off + cnt[0], _WLSZ)

            return lax.fori_loop(0, _ICH // _L, fa_vec, off)

        total = lax.fori_loop(0, B // _ICH, fa_chunk, 0)

        def zero(k, car):
            cnt_s[k] = 0
            return car

        lax.fori_loop(0, _BPWF + 2, zero, 0)

        def hist(p, car):
            e = wl_v[pl.ds(p, _L)][0]
            bl = e & 255
            cnt_s[bl] = cnt_s[bl] + 1
            return car

        lax.fori_loop(0, total, hist, 0)

        def pref(k, acc):
            start_s[k] = acc
            cur_s[k] = acc
            return acc + cnt_s[k]

        lax.fori_loop(0, _BPWF + 2, pref, 0)

        lane0 = jota == 0

        def scat(p, car):
            e = wl_v[pl.ds(p, _L)][0]
            bl = e & 255
            pos = cur_s[bl]
            cur_s[bl] = pos + 1
            plsc.store_scatter(
                wl2_v, [jnp.full((_L,), pos, jnp.int32)],
                jnp.full((_L,), e, jnp.int32), mask=lane0)
            return car

        lax.fori_loop(0, total, scat, 0)

        def chunk(t, car):
            wait_t(t)
            buf = lax.rem(t, 12)
            gs_local = jnp.minimum(t * _CB, nb - _CB)
            p0 = start_s[jnp.minimum(t * _CB, nb)]
            p1 = start_s[jnp.minimum(t * _CB + _CB, nb)]

            def entry(p, car):
                e = wl2_v[pl.ds(p, _L)][0]
                bl = e & 255
                l = (e >> 8) & 127
                i = e >> 15
                lane = jnp.full((_L,), (bl - gs_local) * 128 + l, jnp.int32)
                bufv = jnp.full((_L,), buf, jnp.int32)
                srow = lax.rem(p, _SROWS)

                @pl.when((srow == 0) & (p > 0))
                def _():
                    pltpu.make_async_copy(
                        out_hbm.at[pl.ds(0, _SROWS)], stage_v, sem_o).wait()

                for k in range(NZ // _L):
                    stage_v[srow, pl.ds(k * _L, _L)] = plsc.load_gather(
                        ring_v, [bufv, jota + k * _L, lane])
                pltpu.async_copy(
                    stage_v.at[pl.ds(srow, 1)],
                    out_hbm.at[pl.ds(i, 1)], sem_o)
                return car

            lax.fori_loop(p0, p1, entry, 0)

            @pl.when(t + 12 < nch)
            def _():
                fetch_t(t + 12)

            return car

        lax.fori_loop(0, nch, chunk, 0)

        resid = jnp.where(total > 0, lax.rem(total - 1, _SROWS) + 1, 0)

        def dr(k, car):
            pltpu.make_async_copy(
                out_hbm.at[pl.ds(0, 1)], stage_v.at[pl.ds(0, 1)],
                sem_o).wait()
            return car

        lax.fori_loop(0, resid, dr, 0)

    return gather_kernel


_gather = _make_gather()


def kernel(idx, emb_weight):
    return _gather(idx.astype(jnp.int32), emb_weight.T)


# speedup vs baseline: 1.0323x; 1.0323x over previous
import functools

import jax
import jax.numpy as jnp
from jax import lax
from jax.experimental import pallas as pl
from jax.experimental.pallas import tpu as pltpu, tpu_sc as plsc

N = 1000000
NZ = 64
B = 16384

_info = plsc.get_sparse_core_info()
_NC, _NS, _L = _info.num_cores, _info.num_subcores, _info.num_lanes
_NW = _NC * _NS
_NBLK = (N + 127) // 128
_BPWF = _NBLK // _NW
_REM = _NBLK - _BPWF * _NW
_CB = 1
_ICH = 2048
_SROWS = 64
_WLSZ = 4096


def _make_gather():
    mesh = plsc.VectorSubcoreMesh(core_axis_name="c", subcore_axis_name="s")

    @functools.partial(
        pl.kernel,
        mesh=mesh,
        out_type=jax.ShapeDtypeStruct((B, NZ), jnp.float32),
        scratch_types=[
            pltpu.VMEM((_ICH,), jnp.int32),
            pltpu.VMEM((_WLSZ + _L,), jnp.int32),
            pltpu.VMEM((_WLSZ + _L,), jnp.int32),
            pltpu.VMEM((10, NZ, _CB * 128), jnp.float32),
            pltpu.VMEM((_SROWS, NZ), jnp.float32),
            pltpu.SMEM((_BPWF + 2,), jnp.int32),
            pltpu.SMEM((_BPWF + 2,), jnp.int32),
            pltpu.SMEM((_BPWF + 2,), jnp.int32),
            [pltpu.SemaphoreType.DMA] * 10,
            pltpu.SemaphoreType.DMA,
        ],
        compiler_params=pltpu.CompilerParams(needs_layout_passes=False),
    )
    def gather_kernel(idx_hbm, table_hbm, out_hbm, idxb_v, wl_v, wl2_v,
                      ring_v, stage_v, cnt_s, start_s, cur_s,
                      sems, sem_o):
        wid = lax.axis_index("s") * _NC + lax.axis_index("c")
        lo = _BPWF * wid + jnp.minimum(wid, _REM)
        nb = _BPWF + jnp.where(wid < _REM, 1, 0)
        nch = (nb + _CB - 1) // _CB
        jota = lax.iota(jnp.int32, _L)

        def fetch_t(t):
            gs = lo + jnp.minimum(t * _CB, nb - _CB)
            b0 = pl.multiple_of(gs * 128, 128)
            buf = lax.rem(t, 10)
            for bi, sem in enumerate(sems):
                @pl.when(buf == bi)
                def _(bi=bi, sem=sem):
                    pltpu.async_copy(
                        table_hbm.at[:, pl.ds(b0, _CB * 128)],
                        ring_v.at[bi], sem)

        def wait_t(t):
            buf = lax.rem(t, 10)
            for bi, sem in enumerate(sems):
                @pl.when(buf == bi)
                def _(bi=bi, sem=sem):
                    pltpu.make_async_copy(
                        table_hbm.at[:, pl.ds(0, _CB * 128)],
                        ring_v.at[bi], sem).wait()

        for _t in range(10):
            fetch_t(_t)

        def fa_chunk(c, off):
            pltpu.sync_copy(idx_hbm.at[pl.ds(c * _ICH, _ICH)], idxb_v)

            def fa_vec(g, off):
                v = idxb_v[pl.ds(g * _L, _L)]
                blk = v >> 7
                m = (blk >= lo) & (blk < lo + nb)
                i_vec = c * _ICH + g * _L + jota
                e = (i_vec << 15) | ((v & 127) << 8) | (blk - lo)
                cnt = plsc.all_reduce_population_count(m)
                plsc.store_compressed(wl_v.at[pl.ds(off, _L)], e, mask=m)
                return jnp.minimum(off + cnt[0], _WLSZ)

            return lax.fori_loop(0, _ICH // _L, fa_vec, off)

        total = lax.fori_loop(0, B // _ICH, fa_chunk, 0)

        def zero(k, car):
            cnt_s[k] = 0
            return car

        lax.fori_loop(0, _BPWF + 2, zero, 0)

        def hist(p, car):
            e = wl_v[pl.ds(p, _L)][0]
            bl = e & 255
            cnt_s[bl] = cnt_s[bl] + 1
            return car

        lax.fori_loop(0, total, hist, 0)

        def pref(k, acc):
            start_s[k] = acc
            cur_s[k] = acc
            return acc + cnt_s[k]

        lax.fori_loop(0, _BPWF + 2, pref, 0)

        lane0 = jota == 0

        def scat(p, car):
            e = wl_v[pl.ds(p, _L)][0]
            bl = e & 255
            pos = cur_s[bl]
            cur_s[bl] = pos + 1
            plsc.store_scatter(
                wl2_v, [jnp.full((_L,), pos, jnp.int32)],
                jnp.full((_L,), e, jnp.int32), mask=lane0)
            return car

        lax.fori_loop(0, total, scat, 0)

        def chunk(t, car):
            wait_t(t)
            buf = lax.rem(t, 10)
            gs_local = jnp.minimum(t * _CB, nb - _CB)
            p0 = start_s[jnp.minimum(t * _CB, nb)]
            p1 = start_s[jnp.minimum(t * _CB + _CB, nb)]

            def entry(p, car):
                e = wl2_v[pl.ds(p, _L)][0]
                bl = e & 255
                l = (e >> 8) & 127
                i = e >> 15
                lane = jnp.full((_L,), (bl - gs_local) * 128 + l, jnp.int32)
                bufv = jnp.full((_L,), buf, jnp.int32)
                srow = lax.rem(p, _SROWS)

                @pl.when((srow == 0) & (p > 0))
                def _():
                    pltpu.make_async_copy(
                        out_hbm.at[pl.ds(0, _SROWS)], stage_v, sem_o).wait()

                for k in range(NZ // _L):
                    stage_v[srow, pl.ds(k * _L, _L)] = plsc.load_gather(
                        ring_v, [bufv, jota + k * _L, lane])
                pltpu.async_copy(
                    stage_v.at[pl.ds(srow, 1)],
                    out_hbm.at[pl.ds(i, 1)], sem_o)
                return car

            lax.fori_loop(p0, p1, entry, 0)

            @pl.when(t + 10 < nch)
            def _():
                fetch_t(t + 10)

            return car

        lax.fori_loop(0, nch, chunk, 0)

        resid = jnp.where(total > 0, lax.rem(total - 1, _SROWS) + 1, 0)

        def dr(k, car):
            pltpu.make_async_copy(
                out_hbm.at[pl.ds(0, 1)], stage_v.at[pl.ds(0, 1)],
                sem_o).wait()
            return car

        lax.fori_loop(0, resid, dr, 0)

    return gather_kernel


_gather = _make_gather()


def kernel(idx, emb_weight):
    return _gather(idx.astype(jnp.int32), emb_weight.T)
